# baseline (speedup 1.0000x reference)
import functools

import jax
import jax.numpy as jnp
from jax import lax
from jax.experimental import pallas as pl
from jax.experimental.pallas import tpu as pltpu
from jax.experimental.pallas import tpu_sc as plsc

CHAR_VOCAB = 1000
V_PAD = 1024
WORD_LEN = 16
D_WORD = 128
N_NODES = 16384
NC, NS = 2, 16
NW = NC * NS
G = 4
NPG = N_NODES // G
PPW = NPG // NW
NODES_PER_W = PPW * G
CHUNK_P = 32
NBP = 512


def _sc_counts_body(tok_hbm, ids_hbm, zeros_hbm, cnt_hbm, scl_hbm,
                    idx_v, rows_v, scl_v, buf_a, buf_b, sem_g, sem_a, sem_b):
    wid = lax.axis_index("s") * NC + lax.axis_index("c")
    base_p = wid * PPW
    for g in range(G):
        pltpu.sync_copy(ids_hbm.at[pl.ds(g * NPG + base_p, PPW)],
                        idx_v.at[pl.ds(g * PPW, PPW)])
    ztile_a = pltpu.async_copy(zeros_hbm, buf_a, sem_a)
    ztile_b = pltpu.async_copy(zeros_hbm, buf_b, sem_b)
    gathers = [
        pltpu.async_copy(
            tok_hbm.at[idx_v.at[pl.ds(g * PPW, PPW)]],
            rows_v.at[pl.ds(g * PPW, PPW)],
            sem_g,
        )
        for g in range(G)
    ]
    for h in gathers:
        h.wait()
    ztile_a.wait()
    ztile_b.wait()

    zeros16 = jnp.zeros((16,), jnp.float32)
    iota16 = lax.iota(jnp.int32, 16)
    lane0 = iota16 == 0
    weights = [jnp.full((16,), float(64 ** g), jnp.float32) for g in range(G)]

    n_chunks = PPW // CHUNK_P
    handles = [None, None]
    for k in range(n_chunks):
        b = k % 2
        buf, sem = (buf_a, sem_a) if b == 0 else (buf_b, sem_b)
        if handles[b] is not None:
            handles[b].wait()

            def _rezero(r, _):
                for g in range(G):
                    toks = rows_v[g * PPW + (k - 2) * CHUNK_P + r, :]
                    plsc.store_scatter(buf, [r * V_PAD + toks], zeros16)
                return 0

            lax.fori_loop(0, CHUNK_P, _rezero, 0)

        def _accum(r, _):
            for g in range(G):
                j = k * CHUNK_P + r
                toks = rows_v[g * PPW + j, :]
                plsc.addupdate_scatter(buf, [r * V_PAD + toks], weights[g])
                nonpad = plsc.all_reduce_population_count(toks != 1)
                scale = 1.0 / jnp.maximum(nonpad.astype(jnp.float32), 1.0)
                plsc.store_scatter(scl_v, [jnp.full((16,), g * PPW, jnp.int32) + j],
                                   scale, mask=lane0)
            return 0

        lax.fori_loop(0, CHUNK_P, _accum, 0)
        handles[b] = pltpu.async_copy(
            buf,
            cnt_hbm.at[pl.ds((base_p + k * CHUNK_P) * V_PAD, CHUNK_P * V_PAD)],
            sem)
    scl_dmas = [
        pltpu.async_copy(
            scl_v.at[pl.ds(g * PPW, PPW)],
            scl_hbm.at[pl.ds(g * NPG + base_p, PPW)],
            sem_g,
        )
        for g in range(G)
    ]
    for h in handles:
        if h is not None:
            h.wait()
    for h in scl_dmas:
        h.wait()


def _sc_counts(tokens, lookup_ids, zeros_chunk):
    mesh = plsc.VectorSubcoreMesh(core_axis_name="c", subcore_axis_name="s")
    f = functools.partial(
        pl.kernel,
        mesh=mesh,
        compiler_params=pltpu.CompilerParams(
            use_tc_tiling_on_sc=False, needs_layout_passes=False),
        out_type=(
            jax.ShapeDtypeStruct((NPG * V_PAD,), jnp.float32),
            jax.ShapeDtypeStruct((N_NODES,), jnp.float32),
        ),
        scratch_types=[
            pltpu.VMEM((NODES_PER_W,), jnp.int32),
            pltpu.VMEM((NODES_PER_W, WORD_LEN), jnp.int32),
            pltpu.VMEM((NODES_PER_W,), jnp.float32),
            pltpu.VMEM((CHUNK_P * V_PAD,), jnp.float32),
            pltpu.VMEM((CHUNK_P * V_PAD,), jnp.float32),
            pltpu.SemaphoreType.DMA,
            pltpu.SemaphoreType.DMA,
            pltpu.SemaphoreType.DMA,
        ],
    )(_sc_counts_body)
    return f(tokens, lookup_ids, zeros_chunk)


def _pool_body(cnt_ref, ct_ref, we_ref, be_ref, wc_ref, bc_ref, scl_ref,
               out_ref, c_scr):
    @pl.when(pl.program_id(0) == 0)
    def _build_table():
        e = jnp.dot(ct_ref[...], we_ref[...],
                    preferred_element_type=jnp.float32)
        e = jnp.maximum(e + be_ref[...][None, :], 0.0)
        row = lax.broadcasted_iota(jnp.int32, (V_PAD, 1), 0)
        e = jnp.where(row == 1, 0.0, e)
        c = jnp.dot(e, wc_ref[...], preferred_element_type=jnp.float32)
        c_scr[...] = c.astype(jnp.bfloat16)

    x = cnt_ref[...].reshape(NBP, V_PAD)
    inv64 = jnp.float32(1.0 / 64.0)
    h1 = jnp.floor(x * inv64)
    c0 = x - 64.0 * h1
    h2 = jnp.floor(h1 * inv64)
    c1 = h1 - 64.0 * h2
    h3 = jnp.floor(h2 * inv64)
    c2 = h2 - 64.0 * h3
    c3 = h3
    cmat = c_scr[...]
    bias = bc_ref[...][None, :]
    for g, cg in enumerate((c0, c1, c2, c3)):
        acc = jnp.dot(cg.astype(jnp.bfloat16), cmat,
                      preferred_element_type=jnp.float32)
        out_ref[g] = acc * scl_ref[g] + bias


def _pool(counts, scales3, ct_pad, W_enc, b_enc, W_comp, b_comp):
    return pl.pallas_call(
        _pool_body,
        grid=(NPG // NBP,),
        in_specs=[
            pl.BlockSpec((NBP * V_PAD,), lambda i: (i,)),
            pl.BlockSpec((V_PAD, 64), lambda i: (0, 0)),
            pl.BlockSpec((64, 64), lambda i: (0, 0)),
            pl.BlockSpec((64,), lambda i: (0,)),
            pl.BlockSpec((64, D_WORD), lambda i: (0, 0)),
            pl.BlockSpec((D_WORD,), lambda i: (0,)),
            pl.BlockSpec((G, NBP, 1), lambda i: (0, i, 0)),
        ],
        out_specs=pl.BlockSpec((G, NBP, D_WORD), lambda i: (0, i, 0)),
        out_shape=jax.ShapeDtypeStruct((G, NPG, D_WORD), jnp.float32),
        scratch_shapes=[pltpu.VMEM((V_PAD, D_WORD), jnp.bfloat16)],
    )(counts, ct_pad, W_enc, b_enc, W_comp, b_comp, scales3)


def kernel(distinct_word_tokens, lookup_ids, char_table, W_enc, b_enc, W_comp, b_comp):
    ct_pad = jnp.pad(char_table, ((0, V_PAD - CHAR_VOCAB), (0, 0)))
    zeros_chunk = jnp.zeros((CHUNK_P * V_PAD,), jnp.float32)
    counts, scales = _sc_counts(distinct_word_tokens, lookup_ids, zeros_chunk)
    scales3 = scales.reshape(G, NPG, 1)
    out3 = _pool(counts, scales3, ct_pad, W_enc, b_enc, W_comp, b_comp)
    return out3.reshape(N_NODES, D_WORD)


# speedup vs baseline: 1.0246x; 1.0246x over previous
import functools

import jax
import jax.numpy as jnp
from jax import lax
from jax.experimental import pallas as pl
from jax.experimental.pallas import tpu as pltpu
from jax.experimental.pallas import tpu_sc as plsc

CHAR_VOCAB = 1000
V_PAD = 1024
WORD_LEN = 16
D_WORD = 128
N_NODES = 16384
NC, NS = 2, 16
NW = NC * NS
G = 4
NPG = N_NODES // G
PPW = NPG // NW
NODES_PER_W = PPW * G
CHUNK_P = 32
NBP = 512


def _sc_counts_body(tok_hbm, ids_hbm, zeros_hbm, cnt_hbm,
                    idx_v, rows_v, buf_a, buf_b, sem_g, sem_a, sem_b):
    wid = lax.axis_index("s") * NC + lax.axis_index("c")
    base_p = wid * PPW
    for g in range(G):
        pltpu.sync_copy(ids_hbm.at[pl.ds(g * NPG + base_p, PPW)],
                        idx_v.at[pl.ds(g * PPW, PPW)])
    ztile_a = pltpu.async_copy(zeros_hbm, buf_a, sem_a)
    ztile_b = pltpu.async_copy(zeros_hbm, buf_b, sem_b)
    gathers = [
        pltpu.async_copy(
            tok_hbm.at[idx_v.at[pl.ds(g * PPW, PPW)]],
            rows_v.at[pl.ds(g * PPW, PPW)],
            sem_g,
        )
        for g in range(G)
    ]
    for h in gathers:
        h.wait()
    ztile_a.wait()
    ztile_b.wait()

    zeros16 = jnp.zeros((16,), jnp.float32)
    weights = [jnp.full((16,), float(64 ** g), jnp.float32) for g in range(G)]

    n_chunks = PPW // CHUNK_P
    handles = [None, None]
    for k in range(n_chunks):
        b = k % 2
        buf, sem = (buf_a, sem_a) if b == 0 else (buf_b, sem_b)
        if handles[b] is not None:
            handles[b].wait()

            def _rezero(r, _):
                for g in range(G):
                    toks = rows_v[g * PPW + (k - 2) * CHUNK_P + r, :]
                    plsc.store_scatter(buf, [r * V_PAD + toks], zeros16)
                return 0

            lax.fori_loop(0, CHUNK_P, _rezero, 0)

        def _accum(r, _):
            for g in range(G):
                j = k * CHUNK_P + r
                toks = rows_v[g * PPW + j, :]
                plsc.addupdate_scatter(buf, [r * V_PAD + toks], weights[g])
            return 0

        lax.fori_loop(0, CHUNK_P, _accum, 0)
        handles[b] = pltpu.async_copy(
            buf,
            cnt_hbm.at[pl.ds((base_p + k * CHUNK_P) * V_PAD, CHUNK_P * V_PAD)],
            sem)
    for h in handles:
        if h is not None:
            h.wait()


def _sc_counts(tokens, lookup_ids, zeros_chunk):
    mesh = plsc.VectorSubcoreMesh(core_axis_name="c", subcore_axis_name="s")
    f = functools.partial(
        pl.kernel,
        mesh=mesh,
        compiler_params=pltpu.CompilerParams(
            use_tc_tiling_on_sc=False, needs_layout_passes=False),
        out_type=jax.ShapeDtypeStruct((NPG * V_PAD,), jnp.float32),
        scratch_types=[
            pltpu.VMEM((NODES_PER_W,), jnp.int32),
            pltpu.VMEM((NODES_PER_W, WORD_LEN), jnp.int32),
            pltpu.VMEM((CHUNK_P * V_PAD,), jnp.float32),
            pltpu.VMEM((CHUNK_P * V_PAD,), jnp.float32),
            pltpu.SemaphoreType.DMA,
            pltpu.SemaphoreType.DMA,
            pltpu.SemaphoreType.DMA,
        ],
    )(_sc_counts_body)
    return f(tokens, lookup_ids, zeros_chunk)


def _pool_body(cnt_ref, ct_ref, we_ref, be_ref, wc_ref, bc_ref,
               out_ref, c_scr):
    @pl.when(pl.program_id(0) == 0)
    def _build_table():
        e = jnp.dot(ct_ref[...], we_ref[...],
                    preferred_element_type=jnp.float32)
        e = jnp.maximum(e + be_ref[...][None, :], 0.0)
        row = lax.broadcasted_iota(jnp.int32, (V_PAD, 1), 0)
        e = jnp.where(row == 1, 0.0, e)
        c = jnp.dot(e, wc_ref[...], preferred_element_type=jnp.float32)
        c_scr[...] = c.astype(jnp.bfloat16)

    x = cnt_ref[...].reshape(NBP, V_PAD)
    inv64 = jnp.float32(1.0 / 64.0)
    h1 = jnp.floor(x * inv64)
    c0 = x - 64.0 * h1
    h2 = jnp.floor(h1 * inv64)
    c1 = h1 - 64.0 * h2
    h3 = jnp.floor(h2 * inv64)
    c2 = h2 - 64.0 * h3
    c3 = h3
    cmat = c_scr[...]
    bias = bc_ref[...][None, :]
    for g, cg in enumerate((c0, c1, c2, c3)):
        npad = cg[:, 1:2]
        scale = 1.0 / jnp.maximum(16.0 - npad, 1.0)
        acc = jnp.dot(cg.astype(jnp.bfloat16), cmat,
                      preferred_element_type=jnp.float32)
        out_ref[g] = acc * scale + bias


def _pool(counts, ct_pad, W_enc, b_enc, W_comp, b_comp):
    return pl.pallas_call(
        _pool_body,
        grid=(NPG // NBP,),
        in_specs=[
            pl.BlockSpec((NBP * V_PAD,), lambda i: (i,)),
            pl.BlockSpec((V_PAD, 64), lambda i: (0, 0)),
            pl.BlockSpec((64, 64), lambda i: (0, 0)),
            pl.BlockSpec((64,), lambda i: (0,)),
            pl.BlockSpec((64, D_WORD), lambda i: (0, 0)),
            pl.BlockSpec((D_WORD,), lambda i: (0,)),
        ],
        out_specs=pl.BlockSpec((G, NBP, D_WORD), lambda i: (0, i, 0)),
        out_shape=jax.ShapeDtypeStruct((G, NPG, D_WORD), jnp.float32),
        scratch_shapes=[pltpu.VMEM((V_PAD, D_WORD), jnp.bfloat16)],
    )(counts, ct_pad, W_enc, b_enc, W_comp, b_comp)


def kernel(distinct_word_tokens, lookup_ids, char_table, W_enc, b_enc, W_comp, b_comp):
    ct_pad = jnp.pad(char_table, ((0, V_PAD - CHAR_VOCAB), (0, 0)))
    zeros_chunk = jnp.zeros((CHUNK_P * V_PAD,), jnp.float32)
    counts = _sc_counts(distinct_word_tokens, lookup_ids, zeros_chunk)
    out3 = _pool(counts, ct_pad, W_enc, b_enc, W_comp, b_comp)
    return out3.reshape(N_NODES, D_WORD)


# speedup vs baseline: 1.0292x; 1.0046x over previous
import functools

import jax
import jax.numpy as jnp
from jax import lax
from jax.experimental import pallas as pl
from jax.experimental.pallas import tpu as pltpu
from jax.experimental.pallas import tpu_sc as plsc

CHAR_VOCAB = 1000
V_PAD = 1024
WORD_LEN = 16
D_WORD = 128
N_NODES = 16384
NC, NS = 2, 16
NW = NC * NS
G = 4
NPG = N_NODES // G
PPW = NPG // NW
NODES_PER_W = PPW * G
CHUNK_P = 32
NBP = 512


def _sc_counts_body(tok_hbm, ids_hbm, zeros_hbm, cnt_hbm,
                    idx_v, rows_v, buf_a, buf_b, sem_g, sem_a, sem_b):
    wid = lax.axis_index("s") * NC + lax.axis_index("c")
    base_p = wid * PPW
    for g in range(G):
        pltpu.sync_copy(ids_hbm.at[pl.ds(g * NPG + base_p, PPW)],
                        idx_v.at[pl.ds(g * PPW, PPW)])
    ztile_a = pltpu.async_copy(zeros_hbm, buf_a, sem_a)
    ztile_b = pltpu.async_copy(zeros_hbm, buf_b, sem_b)
    gathers = [
        pltpu.async_copy(
            tok_hbm.at[idx_v.at[pl.ds(g * PPW, PPW)]],
            rows_v.at[pl.ds(g * PPW, PPW)],
            sem_g,
        )
        for g in range(G)
    ]
    for h in gathers:
        h.wait()
    ztile_a.wait()
    ztile_b.wait()

    zeros16 = jnp.zeros((16,), jnp.float32)
    weights = [jnp.full((16,), float(64 ** g), jnp.float32) for g in range(G)]

    n_chunks = PPW // CHUNK_P
    handles = [None, None]
    for k in range(n_chunks):
        b = k % 2
        buf, sem = (buf_a, sem_a) if b == 0 else (buf_b, sem_b)
        if handles[b] is not None:
            handles[b].wait()

            def _rezero(r, _):
                for g in range(G):
                    toks = rows_v[g * PPW + (k - 2) * CHUNK_P + r, :]
                    plsc.store_scatter(buf, [r * V_PAD + toks], zeros16)
                return 0

            lax.fori_loop(0, CHUNK_P, _rezero, 0)

        def _accum(r, _):
            for g in range(G):
                j = k * CHUNK_P + r
                toks = rows_v[g * PPW + j, :]
                plsc.addupdate_scatter(buf, [r * V_PAD + toks], weights[g])
            return 0

        lax.fori_loop(0, CHUNK_P, _accum, 0)
        handles[b] = pltpu.async_copy(
            buf,
            cnt_hbm.at[pl.ds((base_p + k * CHUNK_P) * V_PAD, CHUNK_P * V_PAD)],
            sem)
    for h in handles:
        if h is not None:
            h.wait()


def _sc_counts(tokens, lookup_ids, zeros_chunk):
    mesh = plsc.VectorSubcoreMesh(core_axis_name="c", subcore_axis_name="s")
    f = functools.partial(
        pl.kernel,
        mesh=mesh,
        compiler_params=pltpu.CompilerParams(
            use_tc_tiling_on_sc=False, needs_layout_passes=False),
        out_type=jax.ShapeDtypeStruct((NPG * V_PAD,), jnp.float32),
        scratch_types=[
            pltpu.VMEM((NODES_PER_W,), jnp.int32),
            pltpu.VMEM((NODES_PER_W, WORD_LEN), jnp.int32),
            pltpu.VMEM((CHUNK_P * V_PAD,), jnp.float32),
            pltpu.VMEM((CHUNK_P * V_PAD,), jnp.float32),
            pltpu.SemaphoreType.DMA,
            pltpu.SemaphoreType.DMA,
            pltpu.SemaphoreType.DMA,
        ],
    )(_sc_counts_body)
    return f(tokens, lookup_ids, zeros_chunk)


def _table_body(ct_ref, we_ref, be_ref, wc_ref, c_ref):
    e = jnp.dot(ct_ref[...], we_ref[...], preferred_element_type=jnp.float32)
    e = jnp.maximum(e + be_ref[...][None, :], 0.0)
    row = lax.broadcasted_iota(jnp.int32, (V_PAD, 1), 0)
    e = jnp.where(row == 1, 0.0, e)
    c = jnp.dot(e, wc_ref[...], preferred_element_type=jnp.float32)
    c_ref[...] = c.astype(jnp.bfloat16)


def _comp_table(ct_pad, W_enc, b_enc, W_comp):
    return pl.pallas_call(
        _table_body,
        out_shape=jax.ShapeDtypeStruct((V_PAD, D_WORD), jnp.bfloat16),
    )(ct_pad, W_enc, b_enc, W_comp)


def _pool_body(cnt_ref, c_ref, bc_ref, out_ref):
    x = cnt_ref[...].reshape(NBP, V_PAD)
    inv64 = jnp.float32(1.0 / 64.0)
    h1 = jnp.floor(x * inv64)
    c0 = x - 64.0 * h1
    h2 = jnp.floor(h1 * inv64)
    c1 = h1 - 64.0 * h2
    h3 = jnp.floor(h2 * inv64)
    c2 = h2 - 64.0 * h3
    c3 = h3
    cmat = c_ref[...]
    bias = bc_ref[...][None, :]
    for g, cg in enumerate((c0, c1, c2, c3)):
        npad = cg[:, 1:2]
        scale = 1.0 / jnp.maximum(16.0 - npad, 1.0)
        acc = jnp.dot(cg.astype(jnp.bfloat16), cmat,
                      preferred_element_type=jnp.float32)
        out_ref[g] = acc * scale + bias


def _pool(counts, C, b_comp):
    return pl.pallas_call(
        _pool_body,
        grid=(NPG // NBP,),
        in_specs=[
            pl.BlockSpec((NBP * V_PAD,), lambda i: (i,)),
            pl.BlockSpec((V_PAD, D_WORD), lambda i: (0, 0)),
            pl.BlockSpec((D_WORD,), lambda i: (0,)),
        ],
        out_specs=pl.BlockSpec((G, NBP, D_WORD), lambda i: (0, i, 0)),
        out_shape=jax.ShapeDtypeStruct((G, NPG, D_WORD), jnp.float32),
    )(counts, C, b_comp)


def kernel(distinct_word_tokens, lookup_ids, char_table, W_enc, b_enc, W_comp, b_comp):
    ct_pad = jnp.pad(char_table, ((0, V_PAD - CHAR_VOCAB), (0, 0)))
    zeros_chunk = jnp.zeros((CHUNK_P * V_PAD,), jnp.float32)
    C = _comp_table(ct_pad, W_enc, b_enc, W_comp)
    counts = _sc_counts(distinct_word_tokens, lookup_ids, zeros_chunk)
    out3 = _pool(counts, C, b_comp)
    return out3.reshape(N_NODES, D_WORD)


# speedup vs baseline: 1.0453x; 1.0156x over previous
import functools

import jax
import jax.numpy as jnp
from jax import lax
from jax.experimental import pallas as pl
from jax.experimental.pallas import tpu as pltpu
from jax.experimental.pallas import tpu_sc as plsc

CHAR_VOCAB = 1000
V_PAD = 1024
WORD_LEN = 16
D_WORD = 128
N_NODES = 16384
NC, NS = 2, 16
NW = NC * NS
G = 4
NPG = N_NODES // G
PPW = NPG // NW
NODES_PER_W = PPW * G
CHUNK_P = 32
NBP = 512


def _table_body(ct_ref, we_ref, be_ref, wc_ref, c_ref):
    e = jnp.dot(ct_ref[...], we_ref[...], preferred_element_type=jnp.float32)
    e = jnp.maximum(e + be_ref[...][None, :], 0.0)
    row = lax.broadcasted_iota(jnp.int32, (V_PAD, 1), 0)
    e = jnp.where(row == 1, 0.0, e)
    c = jnp.dot(e, wc_ref[...], preferred_element_type=jnp.float32)
    c_ref[...] = c.astype(jnp.bfloat16)


def _comp_table(ct_pad, W_enc, b_enc, W_comp):
    return pl.pallas_call(
        _table_body,
        out_shape=jax.ShapeDtypeStruct((V_PAD, D_WORD), jnp.bfloat16),
    )(ct_pad, W_enc, b_enc, W_comp)


def _sc_counts_body(tok_hbm, ids_hbm, cnt_hbm, idx_v, rows_v, buf_a, buf_b,
                    sem_g, sem_a, sem_b):
    wid = lax.axis_index("s") * NC + lax.axis_index("c")
    base_p = wid * PPW
    for g in range(G):
        pltpu.sync_copy(ids_hbm.at[pl.ds(g * NPG + base_p, PPW)],
                        idx_v.at[pl.ds(g * PPW, PPW)])
    for g in range(G):
        pltpu.async_copy(
            tok_hbm.at[idx_v.at[pl.ds(g * PPW, PPW)]],
            rows_v.at[pl.ds(g * PPW, PPW)],
            sem_g,
        ).wait()

    zeros16 = jnp.zeros((16,), jnp.float32)
    weights = [jnp.full((16,), float(64 ** g), jnp.float32) for g in range(G)]

    def _zero_vec(i, _):
        off = pl.multiple_of(i * 16, 16)
        buf_a[pl.ds(off, 16)] = zeros16
        buf_b[pl.ds(off, 16)] = zeros16
        return 0

    lax.fori_loop(0, CHUNK_P * V_PAD // 16, _zero_vec, 0)

    n_chunks = PPW // CHUNK_P
    handles = [None, None]
    for k in range(n_chunks):
        b = k % 2
        buf, sem = (buf_a, sem_a) if b == 0 else (buf_b, sem_b)
        if handles[b] is not None:
            handles[b].wait()

            def _rezero(r, _):
                for g in range(G):
                    toks = rows_v[g * PPW + (k - 2) * CHUNK_P + r, :]
                    plsc.store_scatter(buf, [r * V_PAD + toks], zeros16)
                return 0

            lax.fori_loop(0, CHUNK_P, _rezero, 0)

        def _accum(r, _):
            for g in range(G):
                toks = rows_v[g * PPW + k * CHUNK_P + r, :]
                plsc.addupdate_scatter(buf, [r * V_PAD + toks], weights[g])
            return 0

        lax.fori_loop(0, CHUNK_P, _accum, 0)
        handles[b] = pltpu.async_copy(
            buf,
            cnt_hbm.at[pl.ds((base_p + k * CHUNK_P) * V_PAD, CHUNK_P * V_PAD)],
            sem)
    for h in handles:
        if h is not None:
            h.wait()


def _sc_counts(tokens, lookup_ids):
    mesh = plsc.VectorSubcoreMesh(core_axis_name="c", subcore_axis_name="s")
    f = functools.partial(
        pl.kernel,
        mesh=mesh,
        compiler_params=pltpu.CompilerParams(
            use_tc_tiling_on_sc=False, needs_layout_passes=False),
        out_type=jax.ShapeDtypeStruct((NPG * V_PAD,), jnp.float32),
        scratch_types=[
            pltpu.VMEM((NODES_PER_W,), jnp.int32),
            pltpu.VMEM((NODES_PER_W, WORD_LEN), jnp.int32),
            pltpu.VMEM((CHUNK_P * V_PAD,), jnp.float32),
            pltpu.VMEM((CHUNK_P * V_PAD,), jnp.float32),
            pltpu.SemaphoreType.DMA,
            pltpu.SemaphoreType.DMA,
            pltpu.SemaphoreType.DMA,
        ],
    )(_sc_counts_body)
    return f(tokens, lookup_ids)


def _pool_body(cnt_ref, c_ref, bc_ref, out_ref):
    x = cnt_ref[...].reshape(NBP, V_PAD)
    inv64 = jnp.float32(1.0 / 64.0)
    h1 = jnp.floor(x * inv64)
    c0 = x - 64.0 * h1
    h2 = jnp.floor(h1 * inv64)
    c1 = h1 - 64.0 * h2
    h3 = jnp.floor(h2 * inv64)
    c2 = h2 - 64.0 * h3
    c3 = h3
    cmat = c_ref[...]
    bias = bc_ref[...][None, :]
    for g, cg in enumerate((c0, c1, c2, c3)):
        npad = cg[:, 1:2]
        scale = 1.0 / jnp.maximum(16.0 - npad, 1.0)
        acc = jnp.dot(cg.astype(jnp.bfloat16), cmat,
                      preferred_element_type=jnp.float32)
        out_ref[g] = acc * scale + bias


def _pool(counts, C, b_comp):
    return pl.pallas_call(
        _pool_body,
        grid=(NPG // NBP,),
        in_specs=[
            pl.BlockSpec((NBP * V_PAD,), lambda i: (i,)),
            pl.BlockSpec((V_PAD, D_WORD), lambda i: (0, 0)),
            pl.BlockSpec((D_WORD,), lambda i: (0,)),
        ],
        out_specs=pl.BlockSpec((G, NBP, D_WORD), lambda i: (0, i, 0)),
        out_shape=jax.ShapeDtypeStruct((G, NPG, D_WORD), jnp.float32),
    )(counts, C, b_comp)


def kernel(distinct_word_tokens, lookup_ids, char_table, W_enc, b_enc, W_comp, b_comp):
    ct_pad = jnp.pad(char_table, ((0, V_PAD - CHAR_VOCAB), (0, 0)))
    C = _comp_table(ct_pad, W_enc, b_enc, W_comp)
    counts = _sc_counts(distinct_word_tokens, lookup_ids)
    out3 = _pool(counts, C, b_comp)
    return out3.reshape(N_NODES, D_WORD)
